# baseline (speedup 1.0000x reference)
import functools

import jax
import jax.numpy as jnp
from jax import lax
from jax.experimental import pallas as pl
from jax.experimental.pallas import tpu as pltpu
from jax.experimental.pallas import tpu_sc as plsc

B, S, D, V = 4, 8192, 1024, 8192
N = B * S
NC, NS = 2, 16
NW = NC * NS
RW = N // NW
CH = 16
NCHUNK = RW // CH
FR, ER = 4, 3
PF = 2
NV = D // 16

_mesh = plsc.VectorSubcoreMesh(core_axis_name="c", subcore_axis_name="s")


@functools.partial(
    pl.kernel,
    out_type=jax.ShapeDtypeStruct((N, D), jnp.float32),
    mesh=_mesh,
    scratch_types=[
        pltpu.VMEM((RW,), jnp.int32),
        pltpu.VMEM((FR, CH, D), jnp.float32),
        pltpu.VMEM((ER, CH, D), jnp.float32),
        pltpu.SemaphoreType.DMA((FR,)),
        pltpu.SemaphoreType.DMA((ER,)),
        pltpu.SemaphoreType.DMA((FR,)),
    ],
)
def _posemb_kernel(feat_hbm, idx_hbm, table_hbm, out_hbm, idx_all, feat_v,
                   emb_v, fsem, gsem, ssem):
    wid = lax.axis_index("s") * NC + lax.axis_index("c")
    base0 = wid * RW

    pltpu.sync_copy(idx_hbm.at[pl.ds(base0, RW)], idx_all)

    def start_in(g, bf, be):
        base = base0 + g * CH
        pltpu.async_copy(feat_hbm.at[pl.ds(base, CH)], feat_v.at[bf],
                         fsem.at[bf])
        pltpu.async_copy(table_hbm.at[idx_all.at[pl.ds(g * CH, CH)]],
                         emb_v.at[be], gsem.at[be])

    def wait_in(g, bf, be):
        base = base0 + g * CH
        pltpu.make_async_copy(feat_hbm.at[pl.ds(base, CH)], feat_v.at[bf],
                              fsem.at[bf]).wait()
        pltpu.make_async_copy(table_hbm.at[idx_all.at[pl.ds(g * CH, CH)]],
                              emb_v.at[be], gsem.at[be]).wait()

    def start_store(g, bf):
        base = base0 + g * CH
        pltpu.async_copy(feat_v.at[bf], out_hbm.at[pl.ds(base, CH)],
                         ssem.at[bf])

    def wait_store(g, bf):
        base = base0 + g * CH
        pltpu.make_async_copy(feat_v.at[bf], out_hbm.at[pl.ds(base, CH)],
                              ssem.at[bf]).wait()

    def proc(g, bf, be, do_wait_store, do_prefetch):
        wait_in(g, bf, be)
        if do_wait_store:
            wait_store(g - PF, (bf + PF) % FR)
        if do_prefetch:
            start_in(g + PF, (bf + PF) % FR, (be + PF) % ER)

        def add_row(r, c2):
            for c in range(NV):
                sl = pl.ds(c * 16, 16)
                feat_v[bf, r, sl] = feat_v[bf, r, sl] + emb_v[be, r, sl]
            return c2

        lax.fori_loop(0, CH, add_row, 0)
        start_store(g, bf)

    for g in range(PF):
        start_in(g, g, g)
    for g in range(PF):
        proc(g, g, g, False, True)

    PERIOD = 12

    def outer_body(outer, carry):
        for k in range(PERIOD):
            g = PF + outer * PERIOD + k
            proc(g, (PF + k) % FR, (PF + k) % ER, True, True)
        return carry

    lax.fori_loop(0, (NCHUNK - 2 * PF) // PERIOD, outer_body, 0)
    for g in range(NCHUNK - PF, NCHUNK):
        proc(g, g % FR, g % ER, True, False)
    for g in range(NCHUNK - PF, NCHUNK):
        wait_store(g, g % FR)


def kernel(feature, feature_val, table):
    feat = feature.reshape(N, D)
    idx = feature_val.astype(jnp.int32).reshape(N)
    out = _posemb_kernel(feat, idx, table)
    return out.reshape(B, S, D)
